# Initial kernel scaffold; baseline (speedup 1.0000x reference)
import jax
import jax.numpy as jnp
from jax.experimental import pallas as pl


def kernel(input_ids, embed_table, proj_weight):
    raise NotImplementedError("write your pallas kernel here")


# speedup vs baseline: 1.5438x; 1.5438x over previous
import functools

import jax
import jax.numpy as jnp
from jax import lax
from jax.experimental import pallas as pl
from jax.experimental.pallas import tpu as pltpu
from jax.experimental.pallas import tpu_sc as plsc

D_EMB = 64
D_MODEL = 1024

_NC = 2
_NS = 16
_NW = _NC * _NS

_CB = 128


def _gather_body(ids_hbm, table_hbm, out_hbm, idx_v, rows_v, sem, n_per_w):
    wid = lax.axis_index("s") * _NC + lax.axis_index("c")
    base = wid * n_per_w
    pltpu.sync_copy(ids_hbm.at[pl.ds(base, n_per_w)], idx_v)

    def body(i, carry):
        off = i * _CB
        pltpu.async_copy(
            table_hbm.at[idx_v.at[pl.ds(off, _CB)]], rows_v, sem
        ).wait()
        pltpu.sync_copy(rows_v, out_hbm.at[pl.ds(base + off, _CB)])
        return carry

    lax.fori_loop(0, n_per_w // _CB, body, 0)


def _sc_gather(ids, table):
    n = ids.shape[0]
    n_per_w = n // _NW
    mesh = plsc.VectorSubcoreMesh(core_axis_name="c", subcore_axis_name="s")
    k = pl.kernel(
        functools.partial(_gather_body, n_per_w=n_per_w),
        out_type=jax.ShapeDtypeStruct((n, D_EMB), jnp.float32),
        mesh=mesh,
        scratch_types=[
            pltpu.VMEM((n_per_w,), jnp.int32),
            pltpu.VMEM((_CB, D_EMB), jnp.float32),
            pltpu.SemaphoreType.DMA,
        ],
        compiler_params=pltpu.CompilerParams(use_tc_tiling_on_sc=False),
    )
    return k(ids, table)


_BT = 1024


def _proj_body(e_ref, w_ref, o_ref):
    o_ref[...] = lax.dot_general(
        e_ref[...],
        w_ref[...],
        (((1,), (1,)), ((), ())),
        preferred_element_type=jnp.float32,
    )


def _tc_proj(e, w):
    n = e.shape[0]
    return pl.pallas_call(
        _proj_body,
        grid=(n // _BT,),
        in_specs=[
            pl.BlockSpec((_BT, D_EMB), lambda i: (i, 0)),
            pl.BlockSpec((D_MODEL, D_EMB), lambda i: (0, 0)),
        ],
        out_specs=pl.BlockSpec((_BT, D_MODEL), lambda i: (i, 0)),
        out_shape=jax.ShapeDtypeStruct((n, D_MODEL), jnp.float32),
    )(e, w)


def kernel(input_ids, embed_table, proj_weight):
    b, t = input_ids.shape
    ids = input_ids.reshape(-1).astype(jnp.int32)
    e = _sc_gather(ids, embed_table)
    out = _tc_proj(e, proj_weight)
    return out.reshape(b, t, D_MODEL)


# speedup vs baseline: 1.9489x; 1.2624x over previous
import functools

import jax
import jax.numpy as jnp
from jax import lax
from jax.experimental import pallas as pl
from jax.experimental.pallas import tpu as pltpu
from jax.experimental.pallas import tpu_sc as plsc

D_EMB = 64
D_MODEL = 1024

_NC = 2
_NS = 16
_NW = _NC * _NS

_CB2 = 512
_GB = 128


def _gather_body(ids_hbm, table_hbm, out_hbm, idx_v, rows_v, sem, n_per_w):
    wid = lax.axis_index("s") * _NC + lax.axis_index("c")
    base = wid * n_per_w
    pltpu.sync_copy(ids_hbm.at[pl.ds(base, n_per_w)], idx_v)

    def body(i, carry):
        tok = i * (2 * _CB2)
        for half in range(2):
            for j in range(_CB2 // _GB):
                pltpu.async_copy(
                    table_hbm.at[idx_v.at[pl.ds(tok + half * _CB2 + j * _GB, _GB)]],
                    rows_v.at[pl.ds((half * _CB2 + j * _GB), _GB)],
                    sem,
                )
        for half in range(2):
            for j in range(_CB2 // _GB):
                pltpu.make_async_copy(
                    table_hbm.at[idx_v.at[pl.ds(tok + half * _CB2 + j * _GB, _GB)]],
                    rows_v.at[pl.ds((half * _CB2 + j * _GB), _GB)],
                    sem,
                ).wait()
        prow = base // 2 + i * _CB2
        pltpu.sync_copy(
            rows_v.at[pl.ds(0, _CB2)],
            out_hbm.at[pl.ds(prow, _CB2), pl.ds(0, D_EMB)],
        )
        pltpu.sync_copy(
            rows_v.at[pl.ds(_CB2, _CB2)],
            out_hbm.at[pl.ds(prow, _CB2), pl.ds(D_EMB, D_EMB)],
        )
        return carry

    lax.fori_loop(0, n_per_w // (2 * _CB2), body, 0)


def _sc_gather(ids, table):
    n = ids.shape[0]
    n_per_w = n // _NW
    mesh = plsc.VectorSubcoreMesh(core_axis_name="c", subcore_axis_name="s")
    k = pl.kernel(
        functools.partial(_gather_body, n_per_w=n_per_w),
        out_type=jax.ShapeDtypeStruct((n // 2, 2 * D_EMB), jnp.float32),
        mesh=mesh,
        scratch_types=[
            pltpu.VMEM((n_per_w,), jnp.int32),
            pltpu.VMEM((2 * _CB2, D_EMB), jnp.float32),
            pltpu.SemaphoreType.DMA,
        ],
        compiler_params=pltpu.CompilerParams(use_tc_tiling_on_sc=False),
    )
    return k(ids, table)


def _proj_body(e2_ref, w_ref, o_ref):
    p = e2_ref[...]
    w = w_ref[...]
    dn = (((1,), (1,)), ((), ()))
    o_ref[0:_CB2, :] = lax.dot_general(
        p[:, 0:D_EMB], w, dn, preferred_element_type=jnp.float32
    )
    o_ref[_CB2 : 2 * _CB2, :] = lax.dot_general(
        p[:, D_EMB : 2 * D_EMB], w, dn, preferred_element_type=jnp.float32
    )


def _tc_proj(e2, w):
    n2 = e2.shape[0]
    return pl.pallas_call(
        _proj_body,
        grid=(n2 // _CB2,),
        in_specs=[
            pl.BlockSpec((_CB2, 2 * D_EMB), lambda i: (i, 0)),
            pl.BlockSpec((D_MODEL, D_EMB), lambda i: (0, 0)),
        ],
        out_specs=pl.BlockSpec((2 * _CB2, D_MODEL), lambda i: (i, 0)),
        out_shape=jax.ShapeDtypeStruct((2 * n2, D_MODEL), jnp.float32),
    )(e2, w)


def kernel(input_ids, embed_table, proj_weight):
    b, t = input_ids.shape
    ids = input_ids.reshape(-1).astype(jnp.int32)
    e2 = _sc_gather(ids, embed_table)
    out = _tc_proj(e2, proj_weight)
    return out.reshape(b, t, D_MODEL)


# speedup vs baseline: 2.1333x; 1.0946x over previous
import functools

import jax
import jax.numpy as jnp
from jax import lax
from jax.experimental import pallas as pl
from jax.experimental.pallas import tpu as pltpu
from jax.experimental.pallas import tpu_sc as plsc

D_EMB = 64
D_MODEL = 1024

_NC = 2
_NS = 16
_NW = _NC * _NS

_CB2 = 512
_GB = 128


def _gather_body(ids_hbm, table_hbm, out_hbm, idx_v, rows_v, sem, n_per_w):
    wid = lax.axis_index("s") * _NC + lax.axis_index("c")
    base = wid * n_per_w
    pltpu.sync_copy(ids_hbm.at[pl.ds(base, n_per_w)], idx_v)

    def body(i, carry):
        tok = i * (2 * _CB2)
        for half in range(2):
            for j in range(_CB2 // _GB):
                pltpu.async_copy(
                    table_hbm.at[idx_v.at[pl.ds(tok + half * _CB2 + j * _GB, _GB)]],
                    rows_v.at[pl.ds((half * _CB2 + j * _GB), _GB)],
                    sem,
                )
        for half in range(2):
            for j in range(_CB2 // _GB):
                pltpu.make_async_copy(
                    table_hbm.at[idx_v.at[pl.ds(tok + half * _CB2 + j * _GB, _GB)]],
                    rows_v.at[pl.ds((half * _CB2 + j * _GB), _GB)],
                    sem,
                ).wait()
        prow = base // 2 + i * _CB2
        pltpu.sync_copy(
            rows_v.at[pl.ds(0, _CB2)],
            out_hbm.at[pl.ds(prow, _CB2), pl.ds(0, D_EMB)],
        )
        pltpu.sync_copy(
            rows_v.at[pl.ds(_CB2, _CB2)],
            out_hbm.at[pl.ds(prow, _CB2), pl.ds(D_EMB, D_EMB)],
        )
        return carry

    lax.fori_loop(0, n_per_w // (2 * _CB2), body, 0)


def _sc_gather(ids, table):
    n = ids.shape[0]
    n_per_w = n // _NW
    mesh = plsc.VectorSubcoreMesh(core_axis_name="c", subcore_axis_name="s")
    k = pl.kernel(
        functools.partial(_gather_body, n_per_w=n_per_w),
        out_type=jax.ShapeDtypeStruct((n // 2, 2 * D_EMB), jnp.float32),
        mesh=mesh,
        scratch_types=[
            pltpu.VMEM((n_per_w,), jnp.int32),
            pltpu.VMEM((2 * _CB2, D_EMB), jnp.float32),
            pltpu.SemaphoreType.DMA,
        ],
        compiler_params=pltpu.CompilerParams(use_tc_tiling_on_sc=False),
    )
    return k(ids, table)


_G = 2


def _proj_body(e2_ref, w_ref, o_ref):
    p = e2_ref[...]
    w = w_ref[...]
    dn = (((1,), (1,)), ((), ()))
    lo = lax.dot_general(p[:, 0:D_EMB], w, dn, preferred_element_type=jnp.float32)
    hi = lax.dot_general(
        p[:, D_EMB : 2 * D_EMB], w, dn, preferred_element_type=jnp.float32
    )
    for g in range(_G):
        o_ref[2 * g * _CB2 : (2 * g + 1) * _CB2, :] = lo[g * _CB2 : (g + 1) * _CB2]
        o_ref[(2 * g + 1) * _CB2 : (2 * g + 2) * _CB2, :] = hi[g * _CB2 : (g + 1) * _CB2]


def _tc_proj(e2, w):
    n2 = e2.shape[0]
    return pl.pallas_call(
        _proj_body,
        grid=(n2 // (_G * _CB2),),
        in_specs=[
            pl.BlockSpec((_G * _CB2, 2 * D_EMB), lambda i: (i, 0)),
            pl.BlockSpec((D_MODEL, D_EMB), lambda i: (0, 0)),
        ],
        out_specs=pl.BlockSpec((2 * _G * _CB2, D_MODEL), lambda i: (i, 0)),
        out_shape=jax.ShapeDtypeStruct((2 * n2, D_MODEL), jnp.float32),
    )(e2, w)


def kernel(input_ids, embed_table, proj_weight):
    b, t = input_ids.shape
    ids = input_ids.reshape(-1).astype(jnp.int32)
    e2 = _sc_gather(ids, embed_table)
    out = _tc_proj(e2, proj_weight)
    return out.reshape(b, t, D_MODEL)


# speedup vs baseline: 2.1547x; 1.0101x over previous
import functools

import jax
import jax.numpy as jnp
from jax import lax
from jax.experimental import pallas as pl
from jax.experimental.pallas import tpu as pltpu
from jax.experimental.pallas import tpu_sc as plsc

D_EMB = 64
D_MODEL = 1024

_NC = 2
_NS = 16
_NW = _NC * _NS

_CB2 = 512
_GB = 128


def _gather_body(ids_hbm, table_hbm, out_hbm, idx_v, rows_v, sem, n_per_w):
    wid = lax.axis_index("s") * _NC + lax.axis_index("c")
    base = wid * n_per_w
    pltpu.sync_copy(ids_hbm.at[pl.ds(base, n_per_w)], idx_v)

    def body(i, carry):
        tok = i * (2 * _CB2)
        for half in range(2):
            for j in range(_CB2 // _GB):
                pltpu.async_copy(
                    table_hbm.at[idx_v.at[pl.ds(tok + half * _CB2 + j * _GB, _GB)]],
                    rows_v.at[pl.ds((half * _CB2 + j * _GB), _GB)],
                    sem,
                )
        for half in range(2):
            for j in range(_CB2 // _GB):
                pltpu.make_async_copy(
                    table_hbm.at[idx_v.at[pl.ds(tok + half * _CB2 + j * _GB, _GB)]],
                    rows_v.at[pl.ds((half * _CB2 + j * _GB), _GB)],
                    sem,
                ).wait()
        prow = base // 2 + i * _CB2
        pltpu.sync_copy(
            rows_v.at[pl.ds(0, _CB2)],
            out_hbm.at[pl.ds(prow, _CB2), pl.ds(0, D_EMB)],
        )
        pltpu.sync_copy(
            rows_v.at[pl.ds(_CB2, _CB2)],
            out_hbm.at[pl.ds(prow, _CB2), pl.ds(D_EMB, D_EMB)],
        )
        return carry

    lax.fori_loop(0, n_per_w // (2 * _CB2), body, 0)


def _sc_gather(ids, table):
    n = ids.shape[0]
    n_per_w = n // _NW
    mesh = plsc.VectorSubcoreMesh(core_axis_name="c", subcore_axis_name="s")
    k = pl.kernel(
        functools.partial(_gather_body, n_per_w=n_per_w),
        out_type=jax.ShapeDtypeStruct((n // 2, 2 * D_EMB), jnp.float32),
        mesh=mesh,
        scratch_types=[
            pltpu.VMEM((n_per_w,), jnp.int32),
            pltpu.VMEM((2 * _CB2, D_EMB), jnp.float32),
            pltpu.SemaphoreType.DMA,
        ],
        compiler_params=pltpu.CompilerParams(use_tc_tiling_on_sc=False),
    )
    return k(ids, table)


_G = 4


def _proj_body(e2_ref, w_ref, o_ref):
    p = e2_ref[...]
    w = w_ref[...]
    dn = (((1,), (1,)), ((), ()))
    lo = lax.dot_general(p[:, 0:D_EMB], w, dn, preferred_element_type=jnp.float32)
    hi = lax.dot_general(
        p[:, D_EMB : 2 * D_EMB], w, dn, preferred_element_type=jnp.float32
    )
    for g in range(_G):
        o_ref[2 * g * _CB2 : (2 * g + 1) * _CB2, :] = lo[g * _CB2 : (g + 1) * _CB2]
        o_ref[(2 * g + 1) * _CB2 : (2 * g + 2) * _CB2, :] = hi[g * _CB2 : (g + 1) * _CB2]


def _tc_proj(e2, w):
    n2 = e2.shape[0]
    return pl.pallas_call(
        _proj_body,
        grid=(n2 // (_G * _CB2),),
        in_specs=[
            pl.BlockSpec((_G * _CB2, 2 * D_EMB), lambda i: (i, 0)),
            pl.BlockSpec((D_MODEL, D_EMB), lambda i: (0, 0)),
        ],
        out_specs=pl.BlockSpec((2 * _G * _CB2, D_MODEL), lambda i: (i, 0)),
        out_shape=jax.ShapeDtypeStruct((2 * n2, D_MODEL), jnp.float32),
    )(e2, w)


def kernel(input_ids, embed_table, proj_weight):
    b, t = input_ids.shape
    ids = input_ids.reshape(-1).astype(jnp.int32)
    e2 = _sc_gather(ids, embed_table)
    out = _tc_proj(e2, proj_weight)
    return out.reshape(b, t, D_MODEL)
